# baseline (speedup 1.0000x reference)
import functools

import jax
import jax.numpy as jnp
from jax.experimental import pallas as pl
from jax.experimental.pallas import tpu as pltpu

_BLK = 1000


def _fused_gru_kernel(xa_ref, xb_ref, wz_ref, bz_ref, wh_ref, bh_ref, wl_ref,
                      bl_ref, out_ref):
    def body(x, o_ref):
        z = jax.nn.sigmoid(
            jnp.dot(x, wz_ref[...], preferred_element_type=jnp.float32)
            + bz_ref[...])
        ht = jnp.tanh(
            jnp.dot(x, wh_ref[...], preferred_element_type=jnp.float32)
            + bh_ref[...])
        h = jax.nn.relu((1.0 - z) * ht)
        o_ref[...] = (
            jnp.dot(h, wl_ref[...], preferred_element_type=jnp.float32)
            + bl_ref[...])

    body(xa_ref[...], out_ref.at[0])
    body(xb_ref[...], out_ref.at[1])


@functools.partial(jax.jit, static_argnames=())
def kernel(x, edge_index, edge_weight, W_xz, b_xz, W_hz, b_hz, W_xr, b_xr,
           W_hr, b_hr, W_xh, b_xh, W_hh, b_hh, W_lin, b_lin):
    n, f_in = x.shape
    out_len = W_lin.shape[1]
    half = n // 2
    bz = (b_xz + b_hz).reshape(1, -1)
    bh = (b_xh + b_hh).reshape(1, -1)
    bl = b_lin.reshape(1, -1)

    steps = half // _BLK
    out = pl.pallas_call(
        _fused_gru_kernel,
        grid=(steps,),
        in_specs=[
            pl.BlockSpec((_BLK, f_in), lambda i: (i, 0)),
            pl.BlockSpec((_BLK, f_in), lambda i: (i + steps, 0)),
            pl.BlockSpec((f_in, W_xz.shape[1]), lambda i: (0, 0)),
            pl.BlockSpec((1, W_xz.shape[1]), lambda i: (0, 0)),
            pl.BlockSpec((f_in, W_xh.shape[1]), lambda i: (0, 0)),
            pl.BlockSpec((1, W_xh.shape[1]), lambda i: (0, 0)),
            pl.BlockSpec((W_lin.shape[0], out_len), lambda i: (0, 0)),
            pl.BlockSpec((1, out_len), lambda i: (0, 0)),
        ],
        out_specs=pl.BlockSpec((2, _BLK, out_len), lambda i: (0, i, 0)),
        out_shape=jax.ShapeDtypeStruct((2, half, out_len), x.dtype),
        compiler_params=pltpu.CompilerParams(
            dimension_semantics=("parallel",)),
    )(x, x, W_xz, bz, W_xh, bh, W_lin, bl)
    return (out.reshape(n, out_len),)


# speedup vs baseline: 1.0557x; 1.0557x over previous
import functools

import jax
import jax.numpy as jnp
from jax.experimental import pallas as pl
from jax.experimental.pallas import tpu as pltpu

_BLK = 1000
_S = 5


def _fused_gru_kernel(*refs):
    x_refs = refs[:_S]
    wz_ref, bz_ref, wh_ref, bh_ref, wl_ref, bl_ref, out_ref = refs[_S:]
    for s in range(_S):
        x = x_refs[s][...]
        z = jax.nn.sigmoid(
            jnp.dot(x, wz_ref[...], preferred_element_type=jnp.float32)
            + bz_ref[...])
        ht = jnp.tanh(
            jnp.dot(x, wh_ref[...], preferred_element_type=jnp.float32)
            + bh_ref[...])
        h = jax.nn.relu((1.0 - z) * ht)
        out_ref[pl.ds(s * _BLK, _BLK), :] = (
            jnp.dot(h, wl_ref[...], preferred_element_type=jnp.float32)
            + bl_ref[...])


def _x_spec(s, f_in):
    return pl.BlockSpec((_BLK, f_in), lambda i, s=s: (i * _S + s, 0))


@functools.partial(jax.jit, static_argnames=())
def kernel(x, edge_index, edge_weight, W_xz, b_xz, W_hz, b_hz, W_xr, b_xr,
           W_hr, b_hr, W_xh, b_xh, W_hh, b_hh, W_lin, b_lin):
    n, f_in = x.shape
    out_len = W_lin.shape[1]
    bz = (b_xz + b_hz).reshape(1, -1)
    bh = (b_xh + b_hh).reshape(1, -1)
    bl = b_lin.reshape(1, -1)

    steps = n // (_S * _BLK)
    out = pl.pallas_call(
        _fused_gru_kernel,
        grid=(steps,),
        in_specs=[_x_spec(s, f_in) for s in range(_S)] + [
            pl.BlockSpec((f_in, W_xz.shape[1]), lambda i: (0, 0)),
            pl.BlockSpec((1, W_xz.shape[1]), lambda i: (0, 0)),
            pl.BlockSpec((f_in, W_xh.shape[1]), lambda i: (0, 0)),
            pl.BlockSpec((1, W_xh.shape[1]), lambda i: (0, 0)),
            pl.BlockSpec((W_lin.shape[0], out_len), lambda i: (0, 0)),
            pl.BlockSpec((1, out_len), lambda i: (0, 0)),
        ],
        out_specs=pl.BlockSpec((_S * _BLK, out_len), lambda i: (i, 0)),
        out_shape=jax.ShapeDtypeStruct((n, out_len), x.dtype),
        compiler_params=pltpu.CompilerParams(
            dimension_semantics=("parallel",)),
    )(*([x] * _S), W_xz, bz, W_xh, bh, W_lin, bl)
    return (out,)
